# baseline (speedup 1.0000x reference)
import functools

import jax
import jax.numpy as jnp
from jax import lax
from jax.experimental import pallas as pl
from jax.experimental.pallas import tpu as pltpu
from jax.experimental.pallas import tpu_sc as plsc

EMBED_DIM = 64
NUM_WORKERS = 32
CHUNK = 128
NBUF = 2


def _make_lookup(n_seq, n_batch_tiles, vocab):
    assert n_batch_tiles == NUM_WORKERS
    assert n_seq % NBUF == 0
    n_groups = n_seq // NBUF
    mesh = plsc.VectorSubcoreMesh(core_axis_name="c", subcore_axis_name="s")

    @functools.partial(
        pl.kernel,
        mesh=mesh,
        out_type=jax.ShapeDtypeStruct(
            (n_seq, 8, n_batch_tiles, 8, CHUNK), jnp.float32
        ),
        scratch_types=[
            pltpu.VMEM((n_seq, CHUNK), jnp.int32),
            pltpu.VMEM((NBUF * CHUNK, EMBED_DIM), jnp.float32),
            pltpu.VMEM((NBUF, 8, 8, CHUNK), jnp.float32),
            pltpu.SemaphoreType.DMA((NBUF,)),
            pltpu.SemaphoreType.DMA((NBUF,)),
        ],
        compiler_params=pltpu.CompilerParams(
            use_tc_tiling_on_sc=False, needs_layout_passes=False
        ),
    )
    def lookup(table_hbm, idx_hbm, out_hbm, idx_v, raw_v, tp_v, gsem, ssem):
        wid = lax.axis_index("s") * 2 + lax.axis_index("c")
        pltpu.sync_copy(idx_hbm.at[wid], idx_v)

        lane = jax.lax.broadcasted_iota(jnp.int32, (16,), 0)

        def raw_view(b):
            return raw_v.at[pl.ds(b * CHUNK, CHUNK)]

        def gather_start(s, b):
            pltpu.async_copy(table_hbm.at[idx_v.at[s]], raw_view(b), gsem.at[b])

        def gather_wait(s, b):
            pltpu.make_async_copy(
                table_hbm.at[idx_v.at[s]], raw_view(b), gsem.at[b]
            ).wait()

        def store_start(s, b):
            pltpu.async_copy(tp_v.at[b], out_hbm.at[s, slice(None), wid], ssem.at[b])

        def store_wait(s, b):
            pltpu.make_async_copy(
                tp_v.at[b], out_hbm.at[s, slice(None), wid], ssem.at[b]
            ).wait()

        def transpose(b):
            for k in range(8):
                rows = lane + (b * CHUNK + 16 * k)
                for j in range(EMBED_DIM):
                    col = jnp.full((16,), j, jnp.int32)
                    vec = plsc.load_gather(raw_v, [rows, col])
                    tp_v[b, j // 8, j % 8, pl.ds(16 * k, 16)] = vec

        for b in range(NBUF):
            gather_start(b, b)

        def body(g, carry):
            for b in range(NBUF):
                s = g * NBUF + b
                gather_wait(s, b)

                @pl.when(g > 0)
                def _wait_prev():
                    store_wait(s - NBUF, b)

                transpose(b)
                store_start(s, b)

                @pl.when(g < n_groups - 1)
                def _next_gather():
                    gather_start(lax.min(s + NBUF, n_seq - 1), b)

            return carry

        lax.fori_loop(0, n_groups, body, None)
        for b in range(NBUF):
            store_wait((n_groups - 1) * NBUF + b, b)

    return lookup


def kernel(inputs, table):
    batch, seq = inputs.shape
    vocab, dim = table.shape
    n_batch_tiles = batch // CHUNK
    idx3 = (
        inputs.astype(jnp.int32)
        .T.reshape(seq, n_batch_tiles, CHUNK)
        .transpose(1, 0, 2)
    )
    out5 = _make_lookup(seq, n_batch_tiles, vocab)(table, idx3)
    return out5.transpose(2, 4, 0, 1, 3).reshape(batch, seq, dim)


# speedup vs baseline: 1.6600x; 1.6600x over previous
import functools

import jax
import jax.numpy as jnp
from jax import lax
from jax.experimental import pallas as pl
from jax.experimental.pallas import tpu as pltpu
from jax.experimental.pallas import tpu_sc as plsc

EMBED_DIM = 64
NUM_WORKERS = 32
CHUNK = 128
NBUF = 5


def _make_lookup(n_seq, batch):
    assert n_seq % NBUF == 0
    n_groups = n_seq // NBUF
    mesh = plsc.VectorSubcoreMesh(core_axis_name="c", subcore_axis_name="s")

    @functools.partial(
        pl.kernel,
        mesh=mesh,
        out_type=jax.ShapeDtypeStruct((n_seq * batch, EMBED_DIM), jnp.float32),
        scratch_types=[
            pltpu.VMEM((n_seq, CHUNK), jnp.int32),
            pltpu.VMEM((NBUF, CHUNK, EMBED_DIM), jnp.float32),
            pltpu.SemaphoreType.DMA((NBUF,)),
            pltpu.SemaphoreType.DMA((NBUF,)),
        ],
        compiler_params=pltpu.CompilerParams(use_tc_tiling_on_sc=False),
    )
    def lookup(table_hbm, idx_hbm, out_hbm, idx_v, rows_v, gsem, ssem):
        wid = lax.axis_index("s") * 2 + lax.axis_index("c")
        pltpu.sync_copy(idx_hbm.at[wid], idx_v)

        def out_view(s):
            return out_hbm.at[pl.ds(s * batch + wid * CHUNK, CHUNK)]

        def gather_start(s, b):
            pltpu.async_copy(table_hbm.at[idx_v.at[s]], rows_v.at[b], gsem.at[b])

        def gather_wait(s, b):
            pltpu.make_async_copy(
                table_hbm.at[idx_v.at[s]], rows_v.at[b], gsem.at[b]
            ).wait()

        def store_start(s, b):
            pltpu.async_copy(rows_v.at[b], out_view(s), ssem.at[b])

        def store_wait(s, b):
            pltpu.make_async_copy(rows_v.at[b], out_view(s), ssem.at[b]).wait()

        for b in range(NBUF):
            gather_start(b, b)

        def group(g, carry):
            prev = (g - 1) * NBUF
            cur = g * NBUF
            for b in range(NBUF):
                gather_wait(prev + b, b)
                store_start(prev + b, b)
            for b in range(NBUF):
                store_wait(prev + b, b)
                gather_start(cur + b, b)
            return carry

        lax.fori_loop(1, n_groups, group, None)

        last = (n_groups - 1) * NBUF
        for b in range(NBUF):
            gather_wait(last + b, b)
            pltpu.sync_copy(rows_v.at[b], out_view(last + b))

    return lookup


def kernel(inputs, table):
    batch, seq = inputs.shape
    vocab, dim = table.shape
    n_batch_tiles = batch // CHUNK
    idx3 = (
        inputs.astype(jnp.int32)
        .T.reshape(seq, n_batch_tiles, CHUNK)
        .transpose(1, 0, 2)
    )
    out2 = _make_lookup(seq, batch)(table, idx3)
    return out2.reshape(seq, batch, dim).transpose(1, 0, 2)


# speedup vs baseline: 1.6689x; 1.0054x over previous
import functools

import jax
import jax.numpy as jnp
from jax import lax
from jax.experimental import pallas as pl
from jax.experimental.pallas import tpu as pltpu
from jax.experimental.pallas import tpu_sc as plsc

EMBED_DIM = 64
NUM_WORKERS = 32
CHUNK = 128
NBUF = 10


def _make_lookup(n_seq, batch):
    assert n_seq % NBUF == 0
    n_groups = n_seq // NBUF
    mesh = plsc.VectorSubcoreMesh(core_axis_name="c", subcore_axis_name="s")

    @functools.partial(
        pl.kernel,
        mesh=mesh,
        out_type=jax.ShapeDtypeStruct((n_seq * batch, EMBED_DIM), jnp.float32),
        scratch_types=[
            pltpu.VMEM((n_seq, CHUNK), jnp.int32),
            pltpu.VMEM((NBUF, CHUNK, EMBED_DIM), jnp.float32),
            pltpu.SemaphoreType.DMA((NBUF,)),
            pltpu.SemaphoreType.DMA((NBUF,)),
        ],
        compiler_params=pltpu.CompilerParams(use_tc_tiling_on_sc=False),
    )
    def lookup(table_hbm, idx_hbm, out_hbm, idx_v, rows_v, gsem, ssem):
        wid = lax.axis_index("s") * 2 + lax.axis_index("c")
        pltpu.sync_copy(idx_hbm.at[wid], idx_v)

        def out_view(s):
            return out_hbm.at[pl.ds(s * batch + wid * CHUNK, CHUNK)]

        def gather_start(s, b):
            pltpu.async_copy(table_hbm.at[idx_v.at[s]], rows_v.at[b], gsem.at[b])

        def gather_wait(s, b):
            pltpu.make_async_copy(
                table_hbm.at[idx_v.at[s]], rows_v.at[b], gsem.at[b]
            ).wait()

        def store_start(s, b):
            pltpu.async_copy(rows_v.at[b], out_view(s), ssem.at[b])

        def store_wait(s, b):
            pltpu.make_async_copy(rows_v.at[b], out_view(s), ssem.at[b]).wait()

        for b in range(NBUF):
            gather_start(b, b)

        def group(g, carry):
            prev = (g - 1) * NBUF
            cur = g * NBUF
            for b in range(NBUF):
                gather_wait(prev + b, b)
                store_start(prev + b, b)
            for b in range(NBUF):
                store_wait(prev + b, b)
                gather_start(cur + b, b)
            return carry

        lax.fori_loop(1, n_groups, group, None)

        last = (n_groups - 1) * NBUF
        for b in range(NBUF):
            gather_wait(last + b, b)
            pltpu.sync_copy(rows_v.at[b], out_view(last + b))

    return lookup


def kernel(inputs, table):
    batch, seq = inputs.shape
    vocab, dim = table.shape
    n_batch_tiles = batch // CHUNK
    idx3 = (
        inputs.astype(jnp.int32)
        .T.reshape(seq, n_batch_tiles, CHUNK)
        .transpose(1, 0, 2)
    )
    out2 = _make_lookup(seq, batch)(table, idx3)
    return out2.reshape(seq, batch, dim).transpose(1, 0, 2)
